# Initial kernel scaffold; baseline (speedup 1.0000x reference)
import functools

import jax
import jax.numpy as jnp
from jax import lax
from jax.experimental import pallas as pl
from jax.experimental.pallas import tpu as pltpu
from jax.experimental.pallas import tpu_sc as plsc

_VOCAB = 1000
_D = 64
_B = 16384
_CTX = 10
_NEG = 6

_NC = 2
_NS = 16
_L = 16
_NW = _NC * _NS
_BPW = _B // _NW
_G = _BPW // _L

_mesh = plsc.VectorSubcoreMesh(core_axis_name="c", subcore_axis_name="s")


@functools.partial(
    pl.kernel,
    out_type=jax.ShapeDtypeStruct((_B * _NEG,), jnp.float32),
    mesh=_mesh,
    scratch_types=[
        pltpu.VMEM((_VOCAB * _D,), jnp.float32),
        pltpu.VMEM((_BPW * _CTX,), jnp.int32),
        pltpu.VMEM((_BPW * _NEG,), jnp.int32),
        pltpu.VMEM((_BPW * _NEG,), jnp.float32),
    ],
)
def _cbow(iw_hbm, ns_hbm, w_hbm, out_hbm, w_v, iw_v, ns_v, out_v):
    wid = lax.axis_index("s") * _NC + lax.axis_index("c")
    pltpu.sync_copy(w_hbm, w_v)
    pltpu.sync_copy(iw_hbm.at[pl.ds(wid * (_BPW * _CTX), _BPW * _CTX)], iw_v)
    pltpu.sync_copy(ns_hbm.at[pl.ds(wid * (_BPW * _NEG), _BPW * _NEG)], ns_v)

    iota = lax.iota(jnp.int32, (_L,))
    iota_c = iota * _CTX
    iota_n = iota * _NEG

    def body(g, carry):
        base_c = g * (_L * _CTX)
        base_n = g * (_L * _NEG)
        rowx = [plsc.load_gather(iw_v, [iota_c + (base_c + c)]) * _D
                for c in range(_CTX)]
        rown = [plsc.load_gather(ns_v, [iota_n + (base_n + j)]) * _D
                for j in range(_NEG)]
        logits = [jnp.zeros((_L,), jnp.float32) for _ in range(_NEG)]
        for d in range(_D):
            a = plsc.load_gather(w_v, [rowx[0] + d])
            for c in range(1, _CTX):
                a = a + plsc.load_gather(w_v, [rowx[c] + d])
            for j in range(_NEG):
                logits[j] = logits[j] + a * plsc.load_gather(w_v, [rown[j] + d])
        m = logits[0]
        for j in range(1, _NEG):
            m = jnp.maximum(m, logits[j])
        es = [jnp.exp(l - m) for l in logits]
        s = es[0]
        for j in range(1, _NEG):
            s = s + es[j]
        for j in range(_NEG):
            plsc.store_scatter(out_v, [iota_n + (base_n + j)], es[j] / s)
        return carry

    lax.fori_loop(0, _G, body, 0)
    pltpu.sync_copy(out_v, out_hbm.at[pl.ds(wid * (_BPW * _NEG), _BPW * _NEG)])


def kernel(input_words, negative_samples, W):
    out = _cbow(input_words.reshape(-1), negative_samples.reshape(-1),
                W.reshape(-1))
    return out.reshape(_B, _NEG)


# speedup vs baseline: 2.8780x; 2.8780x over previous
import functools

import jax
import jax.numpy as jnp
from jax import lax
from jax.experimental import pallas as pl
from jax.experimental.pallas import tpu as pltpu
from jax.experimental.pallas import tpu_sc as plsc

_VOCAB = 1000
_D = 64
_B = 16384
_CTX = 10
_NEG = 6

_NC = 2
_NS = 16
_L = 16
_NW = _NC * _NS
_BPW = _B // _NW
_G = _BPW // _L

_mesh = plsc.VectorSubcoreMesh(core_axis_name="c", subcore_axis_name="s")


@functools.partial(
    pl.kernel,
    out_type=jax.ShapeDtypeStruct((_B * _NEG,), jnp.float32),
    mesh=_mesh,
    scratch_types=[
        pltpu.VMEM((_VOCAB * _D,), jnp.float32),
        pltpu.VMEM((_BPW * _CTX,), jnp.int32),
        pltpu.VMEM((_BPW * _NEG,), jnp.int32),
        pltpu.VMEM((_BPW * _NEG,), jnp.float32),
    ],
    compiler_params=pltpu.CompilerParams(needs_layout_passes=False),
)
def _cbow(iw_hbm, ns_hbm, w_hbm, out_hbm, w_v, iw_v, ns_v, out_v):
    wid = lax.axis_index("s") * _NC + lax.axis_index("c")
    pltpu.sync_copy(w_hbm, w_v)
    pltpu.sync_copy(iw_hbm.at[pl.ds(wid * (_BPW * _CTX), _BPW * _CTX)], iw_v)
    pltpu.sync_copy(ns_hbm.at[pl.ds(wid * (_BPW * _NEG), _BPW * _NEG)], ns_v)

    iota = lax.iota(jnp.int32, _L)
    iota_c = iota * _CTX
    iota_n = iota * _NEG

    def body(g, carry):
        base_c = g * (_L * _CTX)
        base_n = g * (_L * _NEG)
        rowx = [plsc.load_gather(iw_v, [iota_c + (base_c + c)]) * _D
                for c in range(_CTX)]
        rown = [plsc.load_gather(ns_v, [iota_n + (base_n + j)]) * _D
                for j in range(_NEG)]
        logits = [jnp.zeros((_L,), jnp.float32) for _ in range(_NEG)]
        for d in range(_D):
            a = plsc.load_gather(w_v, [rowx[0] + d])
            for c in range(1, _CTX):
                a = a + plsc.load_gather(w_v, [rowx[c] + d])
            for j in range(_NEG):
                logits[j] = logits[j] + a * plsc.load_gather(w_v, [rown[j] + d])
        m = logits[0]
        for j in range(1, _NEG):
            m = jnp.maximum(m, logits[j])
        es = [jnp.exp(l - m) for l in logits]
        s = es[0]
        for j in range(1, _NEG):
            s = s + es[j]
        for j in range(_NEG):
            plsc.store_scatter(out_v, [iota_n + (base_n + j)], es[j] / s)
        return carry

    lax.fori_loop(0, _G, body, 0)
    pltpu.sync_copy(out_v, out_hbm.at[pl.ds(wid * (_BPW * _NEG), _BPW * _NEG)])


def kernel(input_words, negative_samples, W):
    out = _cbow(input_words.reshape(-1), negative_samples.reshape(-1),
                W.reshape(-1))
    return out.reshape(_B, _NEG)


# speedup vs baseline: 2.8863x; 1.0029x over previous
import functools

import jax
import jax.numpy as jnp
from jax import lax
from jax.experimental import pallas as pl
from jax.experimental.pallas import tpu as pltpu
from jax.experimental.pallas import tpu_sc as plsc

_VOCAB = 1000
_D = 64
_B = 16384
_CTX = 10
_NEG = 6

_NC = 2
_NS = 16
_L = 16
_NW = _NC * _NS
_BPW = _B // _NW
_G = _BPW // _L

_mesh = plsc.VectorSubcoreMesh(core_axis_name="c", subcore_axis_name="s")


@functools.partial(
    pl.kernel,
    out_type=jax.ShapeDtypeStruct((_B * _NEG,), jnp.float32),
    mesh=_mesh,
    scratch_types=[
        pltpu.VMEM((_VOCAB * _D,), jnp.float32),
        pltpu.VMEM((_BPW * _CTX,), jnp.int32),
        pltpu.VMEM((_BPW * _NEG,), jnp.int32),
        pltpu.VMEM((_BPW * _NEG,), jnp.float32),
    ],
    compiler_params=pltpu.CompilerParams(needs_layout_passes=False),
)
def _cbow(iw_hbm, ns_hbm, w_hbm, out_hbm, w_v, iw_v, ns_v, out_v):
    wid = lax.axis_index("s") * _NC + lax.axis_index("c")
    pltpu.sync_copy(w_hbm, w_v)
    pltpu.sync_copy(iw_hbm.at[pl.ds(wid * (_BPW * _CTX), _BPW * _CTX)], iw_v)
    pltpu.sync_copy(ns_hbm.at[pl.ds(wid * (_BPW * _NEG), _BPW * _NEG)], ns_v)

    iota = lax.iota(jnp.int32, _L)
    iota_c = iota * _CTX
    iota_n = iota * _NEG

    @plsc.parallel_loop(0, _G, step=1)
    def body(g):
        base_c = g * (_L * _CTX)
        base_n = g * (_L * _NEG)
        rowx = [plsc.load_gather(iw_v, [iota_c + (base_c + c)]) * _D
                for c in range(_CTX)]
        rown = [plsc.load_gather(ns_v, [iota_n + (base_n + j)]) * _D
                for j in range(_NEG)]
        logits = [jnp.zeros((_L,), jnp.float32) for _ in range(_NEG)]
        for d in range(_D):
            a = plsc.load_gather(w_v, [rowx[0] + d])
            for c in range(1, _CTX):
                a = a + plsc.load_gather(w_v, [rowx[c] + d])
            for j in range(_NEG):
                logits[j] = logits[j] + a * plsc.load_gather(w_v, [rown[j] + d])
        m = logits[0]
        for j in range(1, _NEG):
            m = jnp.maximum(m, logits[j])
        es = [jnp.exp(l - m) for l in logits]
        s = es[0]
        for j in range(1, _NEG):
            s = s + es[j]
        for j in range(_NEG):
            plsc.store_scatter(out_v, [iota_n + (base_n + j)], es[j] / s)

    pltpu.sync_copy(out_v, out_hbm.at[pl.ds(wid * (_BPW * _NEG), _BPW * _NEG)])


def kernel(input_words, negative_samples, W):
    out = _cbow(input_words.reshape(-1), negative_samples.reshape(-1),
                W.reshape(-1))
    return out.reshape(_B, _NEG)
